# Initial kernel scaffold; baseline (speedup 1.0000x reference)
import jax
import jax.numpy as jnp
from jax.experimental import pallas as pl


def kernel(indices, table):
    raise NotImplementedError("write your pallas kernel here")


# speedup vs baseline: 7.3887x; 7.3887x over previous
import functools
import math

import jax
import jax.numpy as jnp
from jax import lax
from jax.experimental import pallas as pl
from jax.experimental.pallas import tpu as pltpu
from jax.experimental.pallas import tpu_sc as plsc

D_MODEL = 128
CHUNK = 128
NBUF = 2
NC = 2
NS = 16
NW = NC * NS
LANES = 16
SCALE = math.sqrt(float(D_MODEL))


@functools.partial(jax.jit, static_argnums=(2,))
def _sc_embed(idx3, table, n_chunks):
    B = NW * n_chunks * CHUNK
    mesh = plsc.VectorSubcoreMesh(core_axis_name="c", subcore_axis_name="s")

    @functools.partial(
        pl.kernel,
        mesh=mesh,
        out_type=jax.ShapeDtypeStruct((B, D_MODEL), jnp.float32),
        scratch_types=[
            pltpu.VMEM((n_chunks, CHUNK), jnp.int32),
            pltpu.VMEM((NBUF, CHUNK, D_MODEL), jnp.float32),
            pltpu.SemaphoreType.DMA,
            pltpu.SemaphoreType.DMA,
        ],
    )
    def k(idx_hbm, table_hbm, out_hbm, idx_v, bufs, g0, g1):
        wid = lax.axis_index("s") * NC + lax.axis_index("c")
        pltpu.sync_copy(idx_hbm.at[wid], idx_v)
        gsem = (g0, g1)
        out_base0 = wid * n_chunks

        for b in range(NBUF):
            pltpu.async_copy(table_hbm.at[idx_v.at[b]], bufs.at[b], gsem[b])

        def group(g, carry):
            for b in range(NBUF):
                j = g * NBUF + b
                buf = bufs.at[b]
                pltpu.make_async_copy(
                    table_hbm.at[idx_v.at[b]], buf, gsem[b]
                ).wait()

                @plsc.parallel_loop(0, CHUNK)
                def _(r):
                    for o in range(0, D_MODEL, LANES):
                        buf[r, pl.ds(o, LANES)] = buf[r, pl.ds(o, LANES)] * SCALE

                pltpu.sync_copy(
                    buf, out_hbm.at[pl.ds((out_base0 + j) * CHUNK, CHUNK)]
                )

                @pl.when(j + NBUF < n_chunks)
                def _():
                    pltpu.async_copy(
                        table_hbm.at[idx_v.at[j + NBUF]], buf, gsem[b]
                    )

            return carry

        lax.fori_loop(0, n_chunks // NBUF, group, 0)

    return k(idx3, table)


def kernel(indices, table):
    S0, S1 = indices.shape
    B = S0 * S1
    n_chunks = B // (NW * CHUNK)
    idx3 = indices.astype(jnp.int32).reshape(NW, n_chunks, CHUNK)
    out = _sc_embed(idx3, table, n_chunks)
    return out.reshape(S0, S1, D_MODEL)


# speedup vs baseline: 7.8449x; 1.0618x over previous
import functools
import math

import jax
import jax.numpy as jnp
from jax import lax
from jax.experimental import pallas as pl
from jax.experimental.pallas import tpu as pltpu
from jax.experimental.pallas import tpu_sc as plsc

D_MODEL = 128
CHUNK = 128
NBUF = 3
NC = 2
NS = 16
NW = NC * NS
LANES = 16
SCALE = math.sqrt(float(D_MODEL))


@functools.partial(jax.jit, static_argnums=(2,))
def _sc_embed(idx3, table, n_chunks):
    B = NW * n_chunks * CHUNK
    mesh = plsc.VectorSubcoreMesh(core_axis_name="c", subcore_axis_name="s")

    @functools.partial(
        pl.kernel,
        mesh=mesh,
        out_type=jax.ShapeDtypeStruct((B, D_MODEL), jnp.float32),
        scratch_types=[
            pltpu.VMEM((n_chunks, CHUNK), jnp.int32),
            pltpu.VMEM((NBUF, CHUNK, D_MODEL), jnp.float32),
            pltpu.SemaphoreType.DMA,
            pltpu.SemaphoreType.DMA,
            pltpu.SemaphoreType.DMA,
            pltpu.SemaphoreType.DMA,
            pltpu.SemaphoreType.DMA,
            pltpu.SemaphoreType.DMA,
        ],
    )
    def k(idx_hbm, table_hbm, out_hbm, idx_v, bufs, g0, g1, g2, s0, s1, s2):
        wid = lax.axis_index("s") * NC + lax.axis_index("c")
        pltpu.sync_copy(idx_hbm.at[wid], idx_v)
        gsem = (g0, g1, g2)
        ssem = (s0, s1, s2)
        out_base0 = wid * n_chunks

        def wait_gather(b):
            pltpu.make_async_copy(
                table_hbm.at[idx_v.at[0]], bufs.at[b], gsem[b]
            ).wait()

        def wait_scatter(b):
            pltpu.make_async_copy(
                bufs.at[b], out_hbm.at[pl.ds(0, CHUNK)], ssem[b]
            ).wait()

        for b in range(NBUF):
            pltpu.async_copy(table_hbm.at[idx_v.at[b]], bufs.at[b], gsem[b])

        def group(g, carry):
            for t in range(NBUF):
                j = g * NBUF + t
                b = t
                b2 = (t - 1) % NBUF
                buf = bufs.at[b]

                @pl.when(j < n_chunks)
                def _():
                    wait_gather(b)

                    @plsc.parallel_loop(0, CHUNK)
                    def _(r):
                        for o in range(0, D_MODEL, LANES):
                            buf[r, pl.ds(o, LANES)] = (
                                buf[r, pl.ds(o, LANES)] * SCALE
                            )

                    pltpu.async_copy(
                        buf,
                        out_hbm.at[pl.ds((out_base0 + j) * CHUNK, CHUNK)],
                        ssem[b],
                    )

                @pl.when((j >= 1) & (j <= n_chunks))
                def _():
                    wait_scatter(b2)

                @pl.when((j >= 1) & (j + 2 < n_chunks))
                def _():
                    pltpu.async_copy(
                        table_hbm.at[idx_v.at[j + 2]], bufs.at[b2], gsem[b2]
                    )

            return carry

        lax.fori_loop(0, (n_chunks + NBUF) // NBUF, group, 0)

    return k(idx3, table)


def kernel(indices, table):
    S0, S1 = indices.shape
    B = S0 * S1
    n_chunks = B // (NW * CHUNK)
    idx3 = indices.astype(jnp.int32).reshape(NW, n_chunks, CHUNK)
    out = _sc_embed(idx3, table, n_chunks)
    return out.reshape(S0, S1, D_MODEL)
